# baseline (speedup 1.0000x reference)
import functools
import jax
import jax.numpy as jnp
from jax import lax
from jax.experimental import pallas as pl
from jax.experimental.pallas import tpu as pltpu, tpu_sc as plsc

_NQ = 8192
_NB = 4
_ND = 2 * _NB
_NW = 32
_QW = (_ND * _NQ) // _NW
_NG = _QW // 16


def _sc_body(qx_h, qy_h, qz_h, kx_h, ky_h, kz_h,
             idx_h, min_h, sum_h,
             qx_v, qy_v, qz_v, kx_v, ky_v, kz_v, min_v, idx_v, sum_v):
    wid = lax.axis_index("s") * 2 + lax.axis_index("c")
    bd = wid // 4
    part = wid % 4
    qbase = wid * _QW
    kbase = bd * _NQ

    pltpu.sync_copy(qx_h.at[pl.ds(qbase, _QW)], qx_v)
    pltpu.sync_copy(qy_h.at[pl.ds(qbase, _QW)], qy_v)
    pltpu.sync_copy(qz_h.at[pl.ds(qbase, _QW)], qz_v)
    pltpu.sync_copy(kx_h.at[pl.ds(kbase, _NQ)], kx_v)
    pltpu.sync_copy(ky_h.at[pl.ds(kbase, _NQ)], ky_v)
    pltpu.sync_copy(kz_h.at[pl.ds(kbase, _NQ)], kz_v)

    def group(g, ssum):
        qx = qx_v[pl.ds(g * 16, 16)]
        qy = qy_v[pl.ds(g * 16, 16)]
        qz = qz_v[pl.ds(g * 16, 16)]

        def step(kg, carry):
            mmin, tid = carry
            kxg = kx_v[pl.ds(kg * 16, 16)]
            kyg = ky_v[pl.ds(kg * 16, 16)]
            kzg = kz_v[pl.ds(kg * 16, 16)]
            jbase = (kg * 16).astype(jnp.float32)
            for i in range(16):
                dx = qx - jnp.broadcast_to(kxg[i:i + 1], (16,))
                dy = qy - jnp.broadcast_to(kyg[i:i + 1], (16,))
                dz = qz - jnp.broadcast_to(kzg[i:i + 1], (16,))
                d = dx * dx + dy * dy + dz * dz
                upd = d < mmin
                mmin = jnp.minimum(mmin, d)
                tid = jnp.where(upd, jbase + jnp.float32(i), tid)
            return mmin, tid

        mmin = jnp.full((16,), jnp.inf, jnp.float32)
        tid = jnp.zeros((16,), jnp.float32)
        mmin, tid = lax.fori_loop(0, _NQ // 16, step, (mmin, tid))
        min_v[pl.ds(g * 16, 16)] = mmin
        idx_v[pl.ds(g * 16, 16)] = tid.astype(jnp.int32)
        return ssum + mmin

    ssum = lax.fori_loop(0, _NG, group, jnp.zeros((16,), jnp.float32))
    sum_v[...] = ssum
    pltpu.sync_copy(min_v, min_h.at[pl.ds(qbase, _QW)])
    pltpu.sync_copy(idx_v, idx_h.at[pl.ds(qbase, _QW)])
    pltpu.sync_copy(sum_v, sum_h.at[pl.ds(wid * 16, 16)])


def _sc_nn(qx, qy, qz, kx, ky, kz):
    mesh = plsc.VectorSubcoreMesh(core_axis_name="c", subcore_axis_name="s")
    f = functools.partial(
        pl.kernel, mesh=mesh,
        out_type=[
            jax.ShapeDtypeStruct((_ND * _NQ,), jnp.int32),
            jax.ShapeDtypeStruct((_ND * _NQ,), jnp.float32),
            jax.ShapeDtypeStruct((_NW * 16,), jnp.float32),
        ],
        scratch_types=[
            pltpu.VMEM((_QW,), jnp.float32),
            pltpu.VMEM((_QW,), jnp.float32),
            pltpu.VMEM((_QW,), jnp.float32),
            pltpu.VMEM((_NQ,), jnp.float32),
            pltpu.VMEM((_NQ,), jnp.float32),
            pltpu.VMEM((_NQ,), jnp.float32),
            pltpu.VMEM((_QW,), jnp.float32),
            pltpu.VMEM((_QW,), jnp.int32),
            pltpu.VMEM((16,), jnp.float32),
        ],
    )(_sc_body)
    return f(qx, qy, qz, kx, ky, kz)


def kernel(pred_points, true_points):
    qs = jnp.concatenate([pred_points, true_points], axis=0)
    ks = jnp.concatenate([true_points, pred_points], axis=0)
    qx = qs[..., 0].reshape(-1)
    qy = qs[..., 1].reshape(-1)
    qz = qs[..., 2].reshape(-1)
    kx = ks[..., 0].reshape(-1)
    ky = ks[..., 1].reshape(-1)
    kz = ks[..., 2].reshape(-1)
    idxs, mins, sums = _sc_nn(qx, qy, qz, kx, ky, kz)
    loss = jnp.sum(sums) / (_NB * _NQ)
    idxs = idxs.reshape(_ND, _NQ)
    return loss, idxs[:_NB], idxs[_NB:]


# speedup vs baseline: 3.1207x; 3.1207x over previous
import functools
import jax
import jax.numpy as jnp
from jax import lax
from jax.experimental import pallas as pl
from jax.experimental.pallas import tpu as pltpu, tpu_sc as plsc

_NQ = 8192
_NB = 4
_ND = 2 * _NB

_ND_SC = 2
_ND_TC = _ND - _ND_SC


_TQ = 128
_NT = _NQ // _TQ
_CK = 128
_NC = _NQ // _CK


def _tc_body(q_ref, k_ref, min_ref, idx_ref, sum_ref):
    t = pl.program_id(1)
    q = q_ref[0]
    qx = q[:, 0:1]
    qy = q[:, 1:2]
    qz = q[:, 2:3]

    def step(j, carry):
        mmin, tid = carry
        k = k_ref[0, :, pl.ds(j * _CK, _CK)]
        dx = qx - k[0:1, :]
        dy = qy - k[1:2, :]
        dz = qz - k[2:3, :]
        d = dx * dx + dy * dy + dz * dz
        upd = d < mmin
        mmin = jnp.minimum(mmin, d)
        tid = jnp.where(upd, j.astype(jnp.float32), tid)
        return mmin, tid

    mmin = jnp.full((_TQ, _CK), jnp.inf, jnp.float32)
    tid = jnp.zeros((_TQ, _CK), jnp.float32)
    mmin, tid = jax.lax.fori_loop(0, _NC, step, (mmin, tid), unroll=8)

    m = jnp.min(mmin, axis=1)
    lane = jax.lax.broadcasted_iota(jnp.int32, (_TQ, _CK), 1).astype(jnp.float32)
    cand = tid * jnp.float32(_CK) + lane
    idxf = jnp.min(jnp.where(mmin == m[:, None], cand, jnp.float32(2 * _NQ)),
                   axis=1)
    min_ref[0, 0, :] = m
    idx_ref[0, 0, :] = idxf.astype(jnp.int32)

    @pl.when(t == 0)
    def _():
        sum_ref[0, 0, :] = jnp.zeros((_TQ,), jnp.float32)

    sum_ref[0, 0, :] += m


def _tc_nn(qs, ks):
    nd = qs.shape[0]
    grid = (nd, _NT)
    mins, idxs, sums = pl.pallas_call(
        _tc_body,
        grid=grid,
        in_specs=[
            pl.BlockSpec((1, _TQ, 3), lambda b, t: (b, t, 0)),
            pl.BlockSpec((1, 3, _NQ), lambda b, t: (b, 0, 0)),
        ],
        out_specs=[
            pl.BlockSpec((1, 1, _TQ), lambda b, t: (b * _NT + t, 0, 0)),
            pl.BlockSpec((1, 1, _TQ), lambda b, t: (b * _NT + t, 0, 0)),
            pl.BlockSpec((1, 1, _TQ), lambda b, t: (b, 0, 0)),
        ],
        out_shape=[
            jax.ShapeDtypeStruct((nd * _NT, 1, _TQ), jnp.float32),
            jax.ShapeDtypeStruct((nd * _NT, 1, _TQ), jnp.int32),
            jax.ShapeDtypeStruct((nd, 1, _TQ), jnp.float32),
        ],
    )(qs, ks)
    return idxs.reshape(nd, _NQ), jnp.sum(sums)


_NW = 32
_QW = (_ND_SC * _NQ) // _NW
_NG = _QW // 16


def _sc_body(qx_h, qy_h, qz_h, kx_h, ky_h, kz_h,
             idx_h, min_h, sum_h,
             qx_v, qy_v, qz_v, kx_v, ky_v, kz_v, min_v, idx_v, sum_v):
    wid = lax.axis_index("s") * 2 + lax.axis_index("c")
    qbase = wid * _QW
    kbase = (qbase // _NQ) * _NQ

    pltpu.sync_copy(qx_h.at[pl.ds(qbase, _QW)], qx_v)
    pltpu.sync_copy(qy_h.at[pl.ds(qbase, _QW)], qy_v)
    pltpu.sync_copy(qz_h.at[pl.ds(qbase, _QW)], qz_v)
    pltpu.sync_copy(kx_h.at[pl.ds(kbase, _NQ)], kx_v)
    pltpu.sync_copy(ky_h.at[pl.ds(kbase, _NQ)], ky_v)
    pltpu.sync_copy(kz_h.at[pl.ds(kbase, _NQ)], kz_v)

    def group(g, ssum):
        qx = qx_v[pl.ds(g * 16, 16)]
        qy = qy_v[pl.ds(g * 16, 16)]
        qz = qz_v[pl.ds(g * 16, 16)]

        def step(kg, carry):
            mmin, tid = carry
            kxg = kx_v[pl.ds(kg * 16, 16)]
            kyg = ky_v[pl.ds(kg * 16, 16)]
            kzg = kz_v[pl.ds(kg * 16, 16)]
            jbase = (kg * 16).astype(jnp.float32)
            for i in range(16):
                dx = qx - jnp.broadcast_to(kxg[i:i + 1], (16,))
                dy = qy - jnp.broadcast_to(kyg[i:i + 1], (16,))
                dz = qz - jnp.broadcast_to(kzg[i:i + 1], (16,))
                d = dx * dx + dy * dy + dz * dz
                upd = d < mmin
                mmin = jnp.minimum(mmin, d)
                tid = jnp.where(upd, jbase + jnp.float32(i), tid)
            return mmin, tid

        mmin = jnp.full((16,), jnp.inf, jnp.float32)
        tid = jnp.zeros((16,), jnp.float32)
        mmin, tid = lax.fori_loop(0, _NQ // 16, step, (mmin, tid))
        min_v[pl.ds(g * 16, 16)] = mmin
        idx_v[pl.ds(g * 16, 16)] = tid.astype(jnp.int32)
        return ssum + mmin

    ssum = lax.fori_loop(0, _NG, group, jnp.zeros((16,), jnp.float32))
    sum_v[...] = ssum
    pltpu.sync_copy(min_v, min_h.at[pl.ds(qbase, _QW)])
    pltpu.sync_copy(idx_v, idx_h.at[pl.ds(qbase, _QW)])
    pltpu.sync_copy(sum_v, sum_h.at[pl.ds(wid * 16, 16)])


def _sc_nn(qx, qy, qz, kx, ky, kz):
    n = qx.shape[0]
    mesh = plsc.VectorSubcoreMesh(core_axis_name="c", subcore_axis_name="s")
    f = functools.partial(
        pl.kernel, mesh=mesh,
        out_type=[
            jax.ShapeDtypeStruct((n,), jnp.int32),
            jax.ShapeDtypeStruct((n,), jnp.float32),
            jax.ShapeDtypeStruct((_NW * 16,), jnp.float32),
        ],
        scratch_types=[
            pltpu.VMEM((_QW,), jnp.float32),
            pltpu.VMEM((_QW,), jnp.float32),
            pltpu.VMEM((_QW,), jnp.float32),
            pltpu.VMEM((_NQ,), jnp.float32),
            pltpu.VMEM((_NQ,), jnp.float32),
            pltpu.VMEM((_NQ,), jnp.float32),
            pltpu.VMEM((_QW,), jnp.float32),
            pltpu.VMEM((_QW,), jnp.int32),
            pltpu.VMEM((16,), jnp.float32),
        ],
    )(_sc_body)
    return f(qx, qy, qz, kx, ky, kz)


def kernel(pred_points, true_points):
    qs = jnp.concatenate([pred_points, true_points], axis=0)
    ks = jnp.concatenate([true_points, pred_points], axis=0)

    idx_tc, sum_tc = _tc_nn(qs[:_ND_TC], ks[:_ND_TC].transpose(0, 2, 1))

    qsc = qs[_ND_TC:]
    ksc = ks[_ND_TC:]
    idx_sc, mins_sc, sums_sc = _sc_nn(
        qsc[..., 0].reshape(-1), qsc[..., 1].reshape(-1),
        qsc[..., 2].reshape(-1),
        ksc[..., 0].reshape(-1), ksc[..., 1].reshape(-1),
        ksc[..., 2].reshape(-1),
    )

    loss = (sum_tc + jnp.sum(sums_sc)) / (_NB * _NQ)
    idxs = jnp.concatenate([idx_tc, idx_sc.reshape(_ND_SC, _NQ)], axis=0)
    return loss, idxs[:_NB], idxs[_NB:]


# speedup vs baseline: 3.4664x; 1.1108x over previous
import functools
import jax
import jax.numpy as jnp
from jax import lax
from jax.experimental import pallas as pl
from jax.experimental.pallas import tpu as pltpu, tpu_sc as plsc

_NQ = 8192
_NB = 4
_ND = 2 * _NB

_TQ = 64
_CK = 128
_NC = _NQ // _CK

_S_SC = 18
_NQ_SC = _S_SC * 128
_NQ_TC = _NQ - _NQ_SC
_NT = _NQ_TC // _TQ


def _tc_body(q_ref, k_ref, min_ref, idx_ref, sum_ref):
    t = pl.program_id(1)
    q = q_ref[0]
    qxb = jnp.broadcast_to(q[:, 0:1], (_TQ, _CK))
    qyb = jnp.broadcast_to(q[:, 1:2], (_TQ, _CK))
    qzb = jnp.broadcast_to(q[:, 2:3], (_TQ, _CK))

    def step(j, carry):
        mmin, tid = carry
        k = k_ref[0, :, pl.ds(j * _CK, _CK)]
        dx = qxb - jnp.broadcast_to(k[0:1, :], (_TQ, _CK))
        dy = qyb - jnp.broadcast_to(k[1:2, :], (_TQ, _CK))
        dz = qzb - jnp.broadcast_to(k[2:3, :], (_TQ, _CK))
        d = dx * dx + dy * dy + dz * dz
        upd = d < mmin
        mmin = jnp.minimum(mmin, d)
        tid = jnp.where(upd, j.astype(jnp.float32), tid)
        return mmin, tid

    mmin = jnp.full((_TQ, _CK), jnp.inf, jnp.float32)
    tid = jnp.zeros((_TQ, _CK), jnp.float32)
    mmin, tid = jax.lax.fori_loop(0, _NC, step, (mmin, tid), unroll=8)

    m = jnp.min(mmin, axis=1)
    lane = jax.lax.broadcasted_iota(jnp.int32, (_TQ, _CK), 1).astype(jnp.float32)
    cand = tid * jnp.float32(_CK) + lane
    idxf = jnp.min(jnp.where(mmin == m[:, None], cand, jnp.float32(2 * _NQ)),
                   axis=1)
    min_ref[0, 0, :] = m
    idx_ref[0, 0, :] = idxf.astype(jnp.int32)

    @pl.when(t == 0)
    def _():
        sum_ref[0, 0, :] = jnp.zeros((_TQ,), jnp.float32)

    sum_ref[0, 0, :] += m


def _tc_nn(qs, ks):
    nd = qs.shape[0]
    grid = (nd, _NT)
    mins, idxs, sums = pl.pallas_call(
        _tc_body,
        grid=grid,
        in_specs=[
            pl.BlockSpec((1, _TQ, 3), lambda b, t: (b, t, 0)),
            pl.BlockSpec((1, 3, _NQ), lambda b, t: (b, 0, 0)),
        ],
        out_specs=[
            pl.BlockSpec((1, 1, _TQ), lambda b, t: (b * _NT + t, 0, 0)),
            pl.BlockSpec((1, 1, _TQ), lambda b, t: (b * _NT + t, 0, 0)),
            pl.BlockSpec((1, 1, _TQ), lambda b, t: (b, 0, 0)),
        ],
        out_shape=[
            jax.ShapeDtypeStruct((nd * _NT, 1, _TQ), jnp.float32),
            jax.ShapeDtypeStruct((nd * _NT, 1, _TQ), jnp.int32),
            jax.ShapeDtypeStruct((nd, 1, _TQ), jnp.float32),
        ],
    )(qs, ks)
    return idxs.reshape(nd, _NQ_TC), jnp.sum(sums)


_NW = 32
_QW = (_ND * _NQ_SC) // _NW
_NG = _QW // 32


def _sc_body(qx_h, qy_h, qz_h, kx_h, ky_h, kz_h,
             idx_h, min_h, sum_h,
             qx_v, qy_v, qz_v, kx_v, ky_v, kz_v, min_v, idx_v, sum_v):
    wid = lax.axis_index("s") * 2 + lax.axis_index("c")
    qbase = wid * _QW
    kbase = (wid // 4) * _NQ

    pltpu.sync_copy(qx_h.at[pl.ds(qbase, _QW)], qx_v)
    pltpu.sync_copy(qy_h.at[pl.ds(qbase, _QW)], qy_v)
    pltpu.sync_copy(qz_h.at[pl.ds(qbase, _QW)], qz_v)
    pltpu.sync_copy(kx_h.at[pl.ds(kbase, _NQ)], kx_v)
    pltpu.sync_copy(ky_h.at[pl.ds(kbase, _NQ)], ky_v)
    pltpu.sync_copy(kz_h.at[pl.ds(kbase, _NQ)], kz_v)

    def group(g, ssum):
        qx0 = qx_v[pl.ds(g * 32, 16)]
        qy0 = qy_v[pl.ds(g * 32, 16)]
        qz0 = qz_v[pl.ds(g * 32, 16)]
        qx1 = qx_v[pl.ds(g * 32 + 16, 16)]
        qy1 = qy_v[pl.ds(g * 32 + 16, 16)]
        qz1 = qz_v[pl.ds(g * 32 + 16, 16)]

        def step(kg, carry):
            m0, t0, m1, t1 = carry
            kxg = kx_v[pl.ds(kg * 16, 16)]
            kyg = ky_v[pl.ds(kg * 16, 16)]
            kzg = kz_v[pl.ds(kg * 16, 16)]
            jbase = (kg * 16).astype(jnp.float32)
            for i in range(16):
                kxb = jnp.broadcast_to(kxg[i:i + 1], (16,))
                kyb = jnp.broadcast_to(kyg[i:i + 1], (16,))
                kzb = jnp.broadcast_to(kzg[i:i + 1], (16,))
                jf = jbase + jnp.float32(i)
                dx0 = qx0 - kxb
                dy0 = qy0 - kyb
                dz0 = qz0 - kzb
                d0 = dx0 * dx0 + dy0 * dy0 + dz0 * dz0
                u0 = d0 < m0
                m0 = jnp.minimum(m0, d0)
                t0 = jnp.where(u0, jf, t0)
                dx1 = qx1 - kxb
                dy1 = qy1 - kyb
                dz1 = qz1 - kzb
                d1 = dx1 * dx1 + dy1 * dy1 + dz1 * dz1
                u1 = d1 < m1
                m1 = jnp.minimum(m1, d1)
                t1 = jnp.where(u1, jf, t1)
            return m0, t0, m1, t1

        inf = jnp.full((16,), jnp.inf, jnp.float32)
        zero = jnp.zeros((16,), jnp.float32)
        m0, t0, m1, t1 = lax.fori_loop(0, _NQ // 16, step,
                                       (inf, zero, inf, zero))
        min_v[pl.ds(g * 32, 16)] = m0
        min_v[pl.ds(g * 32 + 16, 16)] = m1
        idx_v[pl.ds(g * 32, 16)] = t0.astype(jnp.int32)
        idx_v[pl.ds(g * 32 + 16, 16)] = t1.astype(jnp.int32)
        return ssum + m0 + m1

    ssum = lax.fori_loop(0, _NG, group, jnp.zeros((16,), jnp.float32))
    sum_v[...] = ssum
    pltpu.sync_copy(min_v, min_h.at[pl.ds(qbase, _QW)])
    pltpu.sync_copy(idx_v, idx_h.at[pl.ds(qbase, _QW)])
    pltpu.sync_copy(sum_v, sum_h.at[pl.ds(wid * 16, 16)])


def _sc_nn(qx, qy, qz, kx, ky, kz):
    n = qx.shape[0]
    mesh = plsc.VectorSubcoreMesh(core_axis_name="c", subcore_axis_name="s")
    f = functools.partial(
        pl.kernel, mesh=mesh,
        out_type=[
            jax.ShapeDtypeStruct((n,), jnp.int32),
            jax.ShapeDtypeStruct((n,), jnp.float32),
            jax.ShapeDtypeStruct((_NW * 16,), jnp.float32),
        ],
        scratch_types=[
            pltpu.VMEM((_QW,), jnp.float32),
            pltpu.VMEM((_QW,), jnp.float32),
            pltpu.VMEM((_QW,), jnp.float32),
            pltpu.VMEM((_NQ,), jnp.float32),
            pltpu.VMEM((_NQ,), jnp.float32),
            pltpu.VMEM((_NQ,), jnp.float32),
            pltpu.VMEM((_QW,), jnp.float32),
            pltpu.VMEM((_QW,), jnp.int32),
            pltpu.VMEM((16,), jnp.float32),
        ],
    )(_sc_body)
    return f(qx, qy, qz, kx, ky, kz)


def kernel(pred_points, true_points):
    qs = jnp.concatenate([pred_points, true_points], axis=0)
    ks = jnp.concatenate([true_points, pred_points], axis=0)

    idx_tc, sum_tc = _tc_nn(qs[:, :_NQ_TC], ks.transpose(0, 2, 1))

    qsc = qs[:, _NQ_TC:]
    idx_sc, mins_sc, sums_sc = _sc_nn(
        qsc[..., 0].reshape(-1), qsc[..., 1].reshape(-1),
        qsc[..., 2].reshape(-1),
        ks[..., 0].reshape(-1), ks[..., 1].reshape(-1),
        ks[..., 2].reshape(-1),
    )

    loss = (sum_tc + jnp.sum(sums_sc)) / (_NB * _NQ)
    idxs = jnp.concatenate([idx_tc, idx_sc.reshape(_ND, _NQ_SC)], axis=1)
    return loss, idxs[:_NB], idxs[_NB:]
